# Initial kernel scaffold; baseline (speedup 1.0000x reference)
import jax
import jax.numpy as jnp
from jax.experimental import pallas as pl


def kernel(x, edge_index, subgraph_id, W1, b1, a1, W2, b2, a2, fc_w, fc_b):
    raise NotImplementedError("write your pallas kernel here")


# speedup vs baseline: 54.7087x; 54.7087x over previous
import functools

import jax
import jax.numpy as jnp
from jax import lax
from jax.experimental import pallas as pl
from jax.experimental.pallas import tpu as pltpu
from jax.experimental.pallas import tpu_sc as plsc

N = 10000
E = 320000
D = 128
NSUB = 500

NC = 2
NS = 16
NW = NC * NS
CHUNK = 128
NPAD = 10240
RPT = NPAD // NS
DUMP = 10016
HCH = -(-E // (NS * CHUNK))
MCH = -(-E // (NW * CHUNK))

SEGP = 512
PADSEG = SEGP - 1
BLK = 256
NB = NPAD // BLK

def _make_mesh():
    return plsc.VectorSubcoreMesh(core_axis_name="c", subcore_axis_name="s",
                                  num_cores=NC, num_subcores=NS)


@functools.cache
def _get_deg_kernel():
    return functools.partial(
        pl.kernel,
        out_type=jax.ShapeDtypeStruct((2, NPAD), jnp.float32),
        mesh=_make_mesh(),
        scratch_types=[
            pltpu.VMEM((HCH, CHUNK), jnp.int32),
            pltpu.VMEM((CHUNK,), jnp.float32),
            pltpu.VMEM((RPT,), jnp.float32),
            pltpu.VMEM_SHARED((NPAD,), jnp.float32),
        ],
    )(_deg_body)


def _deg_body(idx_hbm, deg_hbm, idx_v, ones_v, buf_v, hist_sh):
    c = lax.axis_index("c")
    s = lax.axis_index("s")
    for i in range(CHUNK // 16):
        ones_v[pl.ds(i * 16, 16)] = jnp.ones((16,), jnp.float32)
    for i in range(RPT // 16):
        buf_v[pl.ds(i * 16, 16)] = jnp.zeros((16,), jnp.float32)
    pltpu.sync_copy(buf_v, hist_sh.at[pl.ds(s * RPT, RPT)])
    plsc.subcore_barrier()
    pltpu.sync_copy(idx_hbm.at[c, s], idx_v)

    def body(j, carry):
        pltpu.sync_copy(ones_v, hist_sh.at[idx_v.at[j]], add=True)
        return carry

    lax.fori_loop(jnp.int32(0), jnp.int32(HCH), body, jnp.int32(0))
    plsc.subcore_barrier()
    pltpu.sync_copy(hist_sh.at[pl.ds(s * RPT, RPT)], buf_v)
    pltpu.sync_copy(buf_v, deg_hbm.at[c, pl.ds(s * RPT, RPT)])


@functools.cache
def _get_agg_kernel():
    return functools.partial(
        pl.kernel,
        out_type=jax.ShapeDtypeStruct((2, NPAD, D), jnp.float32),
        mesh=_make_mesh(),
        scratch_types=[
            pltpu.VMEM((MCH, CHUNK), jnp.int32),
            pltpu.VMEM((MCH, CHUNK), jnp.int32),
            pltpu.VMEM((CHUNK, D), jnp.float32),
            pltpu.VMEM_SHARED((NPAD, D), jnp.float32),
            pltpu.SemaphoreType.DMA,
        ],
    )(_agg_body)


def _agg_body(hs_hbm, srcc_hbm, dstc_hbm, agg_hbm, src_v, dst_v, rows_v,
              acc_sh, sem):
    c = lax.axis_index("c")
    s = lax.axis_index("s")
    w = s * NC + c

    def zrow(i, carry):
        for jj in range(D // 16):
            rows_v[i, pl.ds(jj * 16, 16)] = jnp.zeros((16,), jnp.float32)
        return carry

    lax.fori_loop(jnp.int32(0), jnp.int32(CHUNK), zrow, jnp.int32(0))

    def zacc(r, carry):
        pltpu.sync_copy(rows_v, acc_sh.at[pl.ds(s * RPT + r * CHUNK, CHUNK)])
        return carry

    lax.fori_loop(jnp.int32(0), jnp.int32(RPT // CHUNK), zacc, jnp.int32(0))
    plsc.subcore_barrier()

    pltpu.sync_copy(srcc_hbm.at[w], src_v)
    pltpu.sync_copy(dstc_hbm.at[w], dst_v)

    def body(j, carry):
        pltpu.async_copy(hs_hbm.at[src_v.at[j]], rows_v, sem).wait()
        pltpu.sync_copy(rows_v, acc_sh.at[dst_v.at[j]], add=True)
        return carry

    lax.fori_loop(jnp.int32(0), jnp.int32(MCH), body, jnp.int32(0))
    plsc.subcore_barrier()

    def wout(r, carry):
        pltpu.sync_copy(acc_sh.at[pl.ds(s * RPT + r * CHUNK, CHUNK)], rows_v)
        pltpu.sync_copy(rows_v, agg_hbm.at[c, pl.ds(s * RPT + r * CHUNK, CHUNK)])
        return carry

    lax.fori_loop(jnp.int32(0), jnp.int32(RPT // CHUNK), wout, jnp.int32(0))


def _scale_body(x_ref, do_ref, o_ref):
    d = do_ref[...]
    nsrc = jnp.where(d > 0, lax.rsqrt(d), 0.0)
    o_ref[...] = x_ref[...] * nsrc


def _layer1_body(aggA_ref, aggB_ref, di_ref, do_ref, w_ref, b_ref, a_ref,
                 o_ref):
    di = di_ref[...]
    ndst = jnp.where(di > 0, lax.rsqrt(di), 0.0)
    agg = (aggA_ref[...] + aggB_ref[...]) * ndst
    h = jnp.dot(agg, w_ref[...], preferred_element_type=jnp.float32) + b_ref[...]
    h = jnp.where(h >= 0, h, a_ref[0, 0] * h)
    do = do_ref[...]
    nsrc = jnp.where(do > 0, lax.rsqrt(do), 0.0)
    o_ref[...] = h * nsrc


def _layer2_pool_body(aggA_ref, aggB_ref, di_ref, w_ref, b_ref, a_ref,
                      sid_ref, pooled_ref, cnt_ref):
    i = pl.program_id(0)

    @pl.when(i == 0)
    def _():
        pooled_ref[...] = jnp.zeros_like(pooled_ref)
        cnt_ref[...] = jnp.zeros_like(cnt_ref)

    di = di_ref[...]
    ndst = jnp.where(di > 0, lax.rsqrt(di), 0.0)
    agg = (aggA_ref[...] + aggB_ref[...]) * ndst
    h = jnp.dot(agg, w_ref[...], preferred_element_type=jnp.float32) + b_ref[...]
    h = jnp.where(h >= 0, h, a_ref[0, 0] * h)
    sid = sid_ref[...]
    oh = (sid == lax.broadcasted_iota(jnp.int32, (1, SEGP), 1)).astype(
        jnp.float32)
    pooled_ref[...] += lax.dot_general(
        oh, h, (((0,), (0,)), ((), ())), preferred_element_type=jnp.float32)
    cnt_ref[...] += jnp.sum(oh, axis=0)[:, None]


def _final_body(pooled_ref, cnt_ref, fcw_ref, fcb_ref, g_ref, sid_ref, o_ref,
                row_ref):
    i = pl.program_id(0)

    @pl.when(i == 0)
    def _():
        logits = lax.dot_general(
            pooled_ref[...], fcw_ref[...], (((1,), (1,)), ((), ())),
            preferred_element_type=jnp.float32) + fcb_ref[...]
        present = (cnt_ref[...] > 0).astype(jnp.float32)
        rr = lax.broadcasted_iota(jnp.int32, (SEGP, SEGP), 0)
        cc = lax.broadcasted_iota(jnp.int32, (SEGP, SEGP), 1)
        tril = (cc <= rr).astype(jnp.float32)
        rank = lax.dot_general(
            tril, present, (((1,), (0,)), ((), ())),
            preferred_element_type=jnp.float32) - 1.0
        ranki = rank.astype(jnp.int32)
        gsel_oh = (ranki == lax.broadcasted_iota(jnp.int32, (1, SEGP), 1)
                   ).astype(jnp.float32)
        gsel = lax.dot_general(
            gsel_oh, g_ref[...], (((1,), (0,)), ((), ())),
            preferred_element_type=jnp.float32)
        z = logits + gsel
        m = jnp.max(z, axis=1, keepdims=True)
        e = jnp.exp(z - m)
        ys = e / jnp.sum(e, axis=1, keepdims=True)
        hard0 = (z[:, 0:1] >= z[:, 1:2]).astype(jnp.float32)
        yh = jnp.concatenate([hard0, 1.0 - hard0], axis=1)
        row_ref[...] = (yh - ys) + ys

    sid = sid_ref[...]
    oh = (sid == lax.broadcasted_iota(jnp.int32, (1, SEGP), 1)).astype(
        jnp.float32)
    o_ref[...] = jnp.dot(oh, row_ref[...], preferred_element_type=jnp.float32)


def _row_spec(width):
    return pl.BlockSpec((BLK, width), lambda i: (i, jnp.int32(0)))


def _fix_spec(shape):
    return pl.BlockSpec(shape, lambda i: (jnp.int32(0), jnp.int32(0)))


def kernel(x, edge_index, subgraph_id, W1, b1, a1, W2, b2, a2, fc_w, fc_b):
    src = edge_index[0].astype(jnp.int32)
    dst = edge_index[1].astype(jnp.int32)
    sid = subgraph_id.astype(jnp.int32)

    x_pad = jnp.zeros((NPAD, D), jnp.float32).at[:N].set(
        x.astype(jnp.float32))
    sid_pad = jnp.full((NPAD, 1), PADSEG, jnp.int32).at[:N, 0].set(sid)

    eh = NS * HCH * CHUNK
    srch = jnp.full((eh,), DUMP, jnp.int32).at[:E].set(src)
    dsth = jnp.full((eh,), DUMP, jnp.int32).at[:E].set(dst)
    idxh = jnp.stack([srch.reshape(NS, HCH, CHUNK),
                      dsth.reshape(NS, HCH, CHUNK)])
    em = NW * MCH * CHUNK
    srcm = jnp.full((em,), DUMP, jnp.int32).at[:E].set(src).reshape(
        NW, MCH, CHUNK)
    dstm = jnp.full((em,), DUMP, jnp.int32).at[:E].set(dst).reshape(
        NW, MCH, CHUNK)

    u = jax.random.uniform(jax.random.key(42), (NSUB, 2), minval=1e-10,
                           maxval=1.0)
    g = (-jnp.log(-jnp.log(u))).astype(jnp.float32)
    g_pad = jnp.zeros((SEGP, 2), jnp.float32).at[:NSUB].set(g)

    w1 = W1.astype(jnp.float32)
    w2 = W2.astype(jnp.float32)
    b1r = b1.astype(jnp.float32).reshape(1, D)
    b2r = b2.astype(jnp.float32).reshape(1, D)
    a1r = a1.astype(jnp.float32).reshape(1, 1)
    a2r = a2.astype(jnp.float32).reshape(1, 1)
    fcw = fc_w.astype(jnp.float32)
    fcb = fc_b.astype(jnp.float32).reshape(1, 2)

    deg = _get_deg_kernel()(idxh)
    deg_out = deg[0].reshape(NPAD, 1)
    deg_in = deg[1].reshape(NPAD, 1)

    hs0 = pl.pallas_call(
        _scale_body,
        grid=(NB,),
        in_specs=[_row_spec(D), _row_spec(1)],
        out_specs=_row_spec(D),
        out_shape=jax.ShapeDtypeStruct((NPAD, D), jnp.float32),
    )(x_pad, deg_out)

    agg1 = _get_agg_kernel()(hs0, srcm, dstm)

    hs1 = pl.pallas_call(
        _layer1_body,
        grid=(NB,),
        in_specs=[_row_spec(D), _row_spec(D), _row_spec(1), _row_spec(1),
                  _fix_spec((D, D)), _fix_spec((1, D)), _fix_spec((1, 1))],
        out_specs=_row_spec(D),
        out_shape=jax.ShapeDtypeStruct((NPAD, D), jnp.float32),
    )(agg1[0], agg1[1], deg_in, deg_out, w1, b1r, a1r)

    agg2 = _get_agg_kernel()(hs1, srcm, dstm)

    pooled, cnt = pl.pallas_call(
        _layer2_pool_body,
        grid=(NB,),
        in_specs=[_row_spec(D), _row_spec(D), _row_spec(1),
                  _fix_spec((D, D)), _fix_spec((1, D)), _fix_spec((1, 1)),
                  _row_spec(1)],
        out_specs=[_fix_spec((SEGP, D)), _fix_spec((SEGP, 1))],
        out_shape=[jax.ShapeDtypeStruct((SEGP, D), jnp.float32),
                   jax.ShapeDtypeStruct((SEGP, 1), jnp.float32)],
    )(agg2[0], agg2[1], deg_in, w2, b2r, a2r, sid_pad)

    node_prob = pl.pallas_call(
        _final_body,
        grid=(NB,),
        in_specs=[_fix_spec((SEGP, D)), _fix_spec((SEGP, 1)),
                  _fix_spec((2, D)), _fix_spec((1, 2)), _fix_spec((SEGP, 2)),
                  _row_spec(1)],
        out_specs=_row_spec(2),
        out_shape=jax.ShapeDtypeStruct((NPAD, 2), jnp.float32),
        scratch_shapes=[pltpu.VMEM((SEGP, 2), jnp.float32)],
    )(pooled, cnt, fcw, fcb, g_pad, sid_pad)

    return node_prob[:N].astype(jnp.float64)
